# baseline (speedup 1.0000x reference)
import functools

import jax
import jax.numpy as jnp
from jax import lax
from jax.experimental import pallas as pl
from jax.experimental.pallas import tpu as pltpu
from jax.experimental.pallas import tpu_sc as plsc

_TOP_K = 2
_CAP_FACTOR = 1.0

_info = plsc.get_sparse_core_info()
_NC = _info.num_cores
_NS = _info.num_subcores
_NW = _NC * _NS


def _tc_router(xt, wr):
    T, D = xt.shape
    E = wr.shape[1]
    BT = 512
    wrp = jnp.pad(wr, ((0, 0), (0, 128 - E)))

    def body(x_ref, wr_ref, oi_ref, ow_ref):
        l = jnp.dot(x_ref[...], wr_ref[...], preferred_element_type=jnp.float32)
        lane = lax.broadcasted_iota(jnp.int32, l.shape, 1)
        ninf = jnp.float32(-1e30)
        lm = jnp.where(lane < E, l, ninf)
        m1 = jnp.max(lm, axis=1, keepdims=True)
        i1 = jnp.min(jnp.where(lm == m1, lane, 128), axis=1, keepdims=True)
        lm2 = jnp.where(lane == i1, ninf, lm)
        m2 = jnp.max(lm2, axis=1, keepdims=True)
        i2 = jnp.min(jnp.where(lm2 == m2, lane, 128), axis=1, keepdims=True)
        z = jnp.sum(jnp.exp(lm - m1), axis=1, keepdims=True)
        w1v = 1.0 / z
        w2v = jnp.exp(m2 - m1) / z
        oi_ref[...] = jnp.where(lane == 0, i1, jnp.where(lane == 1, i2, 0))
        ow_ref[...] = jnp.where(lane == 0, w1v, jnp.where(lane == 1, w2v, 0.0))

    oi, ow = pl.pallas_call(
        body,
        grid=(T // BT,),
        in_specs=[
            pl.BlockSpec((BT, D), lambda i: (i, 0)),
            pl.BlockSpec((D, 128), lambda i: (0, 0)),
        ],
        out_specs=[
            pl.BlockSpec((BT, 128), lambda i: (i, 0)),
            pl.BlockSpec((BT, 128), lambda i: (i, 0)),
        ],
        out_shape=[
            jax.ShapeDtypeStruct((T, 128), jnp.int32),
            jax.ShapeDtypeStruct((T, 128), jnp.float32),
        ],
    )(xt, wrp)
    return oi[:, :_TOP_K].reshape(-1), ow[:, :_TOP_K].reshape(-1)


def _prefix16(m):
    iota = lax.iota(jnp.int32, 16)
    c = jnp.where(m, 1, 0)
    for k in (1, 2, 4, 8):
        idx = jnp.maximum(iota - k, 0)
        g = lax.gather(
            c, idx[:, None],
            lax.GatherDimensionNumbers(offset_dims=(), collapsed_slice_dims=(0,),
                                       start_index_map=(0,)),
            slice_sizes=(1,), mode=lax.GatherScatterMode.PROMISE_IN_BOUNDS)
        c = c + jnp.where(iota >= k, g, 0)
    return c


def _sc_keys(s_e, E, cap):
    S = s_e.shape[0]
    NV = S // 16
    R = E * cap
    NT = 16
    nvt = NV // NT
    mesh = plsc.VectorSubcoreMesh(core_axis_name="c", subcore_axis_name="s")

    @functools.partial(
        pl.kernel, mesh=mesh,
        out_type=jax.ShapeDtypeStruct((S,), jnp.int32),
        scratch_types=[
            pltpu.VMEM((S,), jnp.int32),
            pltpu.VMEM((S // NT,), jnp.int32),
        ],
    )
    def keys_k(se_hbm, key_hbm, se_v, key_v):
        role = lax.axis_index("s") * _NC + lax.axis_index("c")

        @pl.when(role < NT)
        def _():
            v0 = role * nvt
            pltpu.sync_copy(se_hbm, se_v)

            def pcnt(v, cnts):
                se = se_v[pl.ds(v * 16, 16)]
                return tuple(
                    cnts[e] + _prefix16(se == e)[15] for e in range(E))

            cnts0 = lax.fori_loop(0, v0, pcnt, (jnp.int32(0),) * E,
                                  unroll=False)

            def body(v, cnts):
                se = se_v[pl.ds(v * 16, 16)]
                keyv = jnp.full((16,), R, jnp.int32)
                new = []
                for e in range(E):
                    m = se == e
                    c = _prefix16(m)
                    pos = cnts[e] + c - 1
                    val = m & (pos < cap)
                    keyv = jnp.where(val, e * cap + pos, keyv)
                    new.append(cnts[e] + c[15])
                key_v[pl.ds((v - v0) * 16, 16)] = keyv
                return tuple(new)

            lax.fori_loop(v0, v0 + nvt, body, cnts0, unroll=False)
            pltpu.sync_copy(key_v, key_hbm.at[pl.ds(v0 * 16, nvt * 16)])

    return keys_k(s_e)


def _sc_dispatch_xg(xt, k0_2d, k1_2d, E, cap):
    T, D = xt.shape
    t_per_w = T // _NW
    TB = 32
    nch = t_per_w // TB
    mesh = plsc.VectorSubcoreMesh(core_axis_name="c", subcore_axis_name="s")

    @functools.partial(
        pl.kernel, mesh=mesh,
        out_type=jax.ShapeDtypeStruct(((E + 1) * cap, D), jnp.float32),
        scratch_types=[
            pltpu.VMEM((nch, TB), jnp.int32),
            pltpu.VMEM((nch, TB), jnp.int32),
            pltpu.VMEM((TB, D), jnp.float32),
            pltpu.SemaphoreType.DMA,
            pltpu.SemaphoreType.DMA,
        ],
    )
    def disp_k(xt_hbm, k0_hbm, k1_hbm, xg_hbm, idx_a, idx_b, buf, sem_a, sem_b):
        wid = lax.axis_index("s") * _NC + lax.axis_index("c")
        base = wid * t_per_w
        pltpu.sync_copy(k0_hbm.at[pl.ds(wid * nch, nch)], idx_a)
        pltpu.sync_copy(k1_hbm.at[pl.ds(wid * nch, nch)], idx_b)

        def chunk(i, carry):
            pltpu.sync_copy(xt_hbm.at[pl.ds(base + i * TB, TB)], buf)
            cp_a = pltpu.async_copy(buf, xg_hbm.at[idx_a.at[i]], sem_a)
            cp_b = pltpu.async_copy(buf, xg_hbm.at[idx_b.at[i]], sem_b)
            cp_a.wait()
            cp_b.wait()
            return carry

        lax.fori_loop(0, nch, chunk, 0, unroll=False)

    return disp_k(xt, k0_2d, k1_2d)


def _ffn(xg, w1, w2, cap):
    D = xg.shape[1]
    E, _, F = w1.shape
    BF = 512
    nf = F // BF

    def body(xg_ref, w1_ref, w2_ref, o_ref):
        e = pl.program_id(0)
        f = pl.program_id(1)

        @pl.when(f == 0)
        def _():
            o_ref[...] = jnp.zeros_like(o_ref)

        @pl.when(e < E)
        def _():
            xgb = xg_ref[...].astype(jnp.bfloat16)
            w1b = w1_ref[0].astype(jnp.bfloat16)
            w2b = w2_ref[0].astype(jnp.bfloat16)
            h = jnp.dot(xgb, w1b, preferred_element_type=jnp.float32)
            h = h * 0.5 * (1.0 + lax.erf(h * 0.7071067811865476))
            o_ref[...] += jnp.dot(h.astype(jnp.bfloat16), w2b,
                                  preferred_element_type=jnp.float32)

    clamp = lambda e: jnp.minimum(e, E - 1)
    return pl.pallas_call(
        body,
        grid=(E + 1, nf),
        in_specs=[
            pl.BlockSpec((cap, D), lambda e, f: (clamp(e), 0)),
            pl.BlockSpec((1, D, BF), lambda e, f: (clamp(e), 0, jnp.where(e < E, f, 0))),
            pl.BlockSpec((1, BF, D), lambda e, f: (clamp(e), jnp.where(e < E, f, 0), 0)),
        ],
        out_specs=pl.BlockSpec((cap, D), lambda e, f: (e, 0)),
        out_shape=jax.ShapeDtypeStruct(((E + 1) * cap, D), jnp.float32),
    )(xg, w1, w2)


def _sc_combine(o, k0, k1, w0, w1):
    T = k0.shape[0]
    D = o.shape[1]
    t_per_w = T // _NW
    TB = 32
    mesh = plsc.VectorSubcoreMesh(core_axis_name="c", subcore_axis_name="s")

    @functools.partial(
        pl.kernel, mesh=mesh,
        out_type=jax.ShapeDtypeStruct((T, D), jnp.float32),
        scratch_types=[
            pltpu.VMEM((t_per_w,), jnp.int32),
            pltpu.VMEM((t_per_w,), jnp.int32),
            pltpu.VMEM((t_per_w,), jnp.float32),
            pltpu.VMEM((t_per_w,), jnp.float32),
            pltpu.VMEM((TB, D), jnp.float32),
            pltpu.VMEM((TB, D), jnp.float32),
            pltpu.VMEM((TB, D), jnp.float32),
            pltpu.SemaphoreType.DMA,
            pltpu.SemaphoreType.DMA,
        ],
    )
    def combine_k(o_hbm, k0_hbm, k1_hbm, w0_hbm, w1_hbm, out_hbm,
                  k0_v, k1_v, w0_v, w1_v, buf_a, buf_b, y_v, sem_a, sem_b):
        wid = lax.axis_index("s") * _NC + lax.axis_index("c")
        base = wid * t_per_w
        pltpu.sync_copy(k0_hbm.at[pl.ds(base, t_per_w)], k0_v)
        pltpu.sync_copy(k1_hbm.at[pl.ds(base, t_per_w)], k1_v)
        pltpu.sync_copy(w0_hbm.at[pl.ds(base, t_per_w)], w0_v)
        pltpu.sync_copy(w1_hbm.at[pl.ds(base, t_per_w)], w1_v)
        dn = lax.GatherDimensionNumbers(offset_dims=(), collapsed_slice_dims=(0,),
                                        start_index_map=(0,))

        def bcast(wv, lane):
            idx = jnp.zeros((16,), jnp.int32) + lane
            return lax.gather(wv, idx[:, None], dn, slice_sizes=(1,),
                              mode=lax.GatherScatterMode.PROMISE_IN_BOUNDS)

        def chunk(i, carry):
            cp_a = pltpu.async_copy(o_hbm.at[k0_v.at[pl.ds(i * TB, TB)]], buf_a, sem_a)
            cp_b = pltpu.async_copy(o_hbm.at[k1_v.at[pl.ds(i * TB, TB)]], buf_b, sem_b)
            cp_a.wait()
            cp_b.wait()

            def per_token(j, c2):
                g16 = (j // 16) * 16
                wva = w0_v[pl.ds(i * TB + g16, 16)]
                wvb = w1_v[pl.ds(i * TB + g16, 16)]
                lane = j - g16
                wa = bcast(wva, lane)
                wb = bcast(wvb, lane)

                def per_lane(g, c3):
                    sl = pl.ds(g * 16, 16)
                    y_v[j, sl] = buf_a[j, sl] * wa + buf_b[j, sl] * wb
                    return c3

                return lax.fori_loop(0, D // 16, per_lane, c2, unroll=4)

            lax.fori_loop(0, TB, per_token, 0, unroll=False)
            pltpu.sync_copy(y_v, out_hbm.at[pl.ds(base + i * TB, TB)])
            return carry

        lax.fori_loop(0, t_per_w // TB, chunk, 0, unroll=False)

    return combine_k(o, k0, k1, w0, w1)


def kernel(x, wr, w1, w2):
    Bq, Sq, D = x.shape
    E = w1.shape[0]
    T = Bq * Sq
    cap = int(_CAP_FACTOR * _TOP_K * T / E)
    xt = jnp.transpose(x, (1, 0, 2)).reshape(T, D)
    s_e, s_w = _tc_router(xt, wr)
    key = _sc_keys(s_e, E, cap)
    k0 = key[0::2]
    k1 = key[1::2]
    w0 = s_w[0::2]
    w1s = s_w[1::2]
    TB = 32
    xg = _sc_dispatch_xg(xt, k0.reshape(T // TB, TB), k1.reshape(T // TB, TB),
                         E, cap)
    o = _ffn(xg, w1, w2, cap)
    y = _sc_combine(o, k0, k1, w0, w1s)
    return jnp.transpose(y.reshape(Sq, Bq, D), (1, 0, 2))


# speedup vs baseline: 1.0100x; 1.0100x over previous
import functools

import jax
import jax.numpy as jnp
from jax import lax
from jax.experimental import pallas as pl
from jax.experimental.pallas import tpu as pltpu
from jax.experimental.pallas import tpu_sc as plsc

_TOP_K = 2
_CAP_FACTOR = 1.0

_info = plsc.get_sparse_core_info()
_NC = _info.num_cores
_NS = _info.num_subcores
_NW = _NC * _NS


def _tc_router(xt, wr):
    T, D = xt.shape
    E = wr.shape[1]
    BT = 512
    wrp = jnp.pad(wr, ((0, 0), (0, 128 - E)))

    def body(x_ref, wr_ref, oi_ref, ow_ref):
        l = jnp.dot(x_ref[...], wr_ref[...], preferred_element_type=jnp.float32)
        lane = lax.broadcasted_iota(jnp.int32, l.shape, 1)
        ninf = jnp.float32(-1e30)
        lm = jnp.where(lane < E, l, ninf)
        m1 = jnp.max(lm, axis=1, keepdims=True)
        i1 = jnp.min(jnp.where(lm == m1, lane, 128), axis=1, keepdims=True)
        lm2 = jnp.where(lane == i1, ninf, lm)
        m2 = jnp.max(lm2, axis=1, keepdims=True)
        i2 = jnp.min(jnp.where(lm2 == m2, lane, 128), axis=1, keepdims=True)
        z = jnp.sum(jnp.exp(lm - m1), axis=1, keepdims=True)
        w1v = 1.0 / z
        w2v = jnp.exp(m2 - m1) / z
        oi_ref[...] = jnp.where(lane == 0, i1, jnp.where(lane == 1, i2, 0))
        ow_ref[...] = jnp.where(lane == 0, w1v, jnp.where(lane == 1, w2v, 0.0))

    oi, ow = pl.pallas_call(
        body,
        grid=(T // BT,),
        in_specs=[
            pl.BlockSpec((BT, D), lambda i: (i, 0)),
            pl.BlockSpec((D, 128), lambda i: (0, 0)),
        ],
        out_specs=[
            pl.BlockSpec((BT, 128), lambda i: (i, 0)),
            pl.BlockSpec((BT, 128), lambda i: (i, 0)),
        ],
        out_shape=[
            jax.ShapeDtypeStruct((T, 128), jnp.int32),
            jax.ShapeDtypeStruct((T, 128), jnp.float32),
        ],
    )(xt, wrp)
    return oi[:, :_TOP_K].reshape(-1), ow[:, :_TOP_K].reshape(-1)


def _prefix16(m):
    iota = lax.iota(jnp.int32, 16)
    c = jnp.where(m, 1, 0)
    for k in (1, 2, 4, 8):
        idx = jnp.maximum(iota - k, 0)
        g = lax.gather(
            c, idx[:, None],
            lax.GatherDimensionNumbers(offset_dims=(), collapsed_slice_dims=(0,),
                                       start_index_map=(0,)),
            slice_sizes=(1,), mode=lax.GatherScatterMode.PROMISE_IN_BOUNDS)
        c = c + jnp.where(iota >= k, g, 0)
    return c


def _sc_keys(s_e, E, cap):
    S = s_e.shape[0]
    NV = S // 16
    R = E * cap
    NT = 16
    nvt = NV // NT
    mesh = plsc.VectorSubcoreMesh(core_axis_name="c", subcore_axis_name="s")

    @functools.partial(
        pl.kernel, mesh=mesh,
        out_type=jax.ShapeDtypeStruct((S,), jnp.int32),
        scratch_types=[
            pltpu.VMEM((S,), jnp.int32),
            pltpu.VMEM((S // NT,), jnp.int32),
        ],
    )
    def keys_k(se_hbm, key_hbm, se_v, key_v):
        role = lax.axis_index("s") * _NC + lax.axis_index("c")

        @pl.when(role < NT)
        def _():
            v0 = role * nvt
            pltpu.sync_copy(se_hbm, se_v)

            def pcnt(v, cnts):
                se = se_v[pl.ds(v * 16, 16)]
                return tuple(
                    cnts[e] + _prefix16(se == e)[15] for e in range(E))

            cnts0 = lax.fori_loop(0, v0, pcnt, (jnp.int32(0),) * E,
                                  unroll=False)

            def body(v, cnts):
                se = se_v[pl.ds(v * 16, 16)]
                keyv = jnp.full((16,), R, jnp.int32)
                new = []
                for e in range(E):
                    m = se == e
                    c = _prefix16(m)
                    pos = cnts[e] + c - 1
                    val = m & (pos < cap)
                    keyv = jnp.where(val, e * cap + pos, keyv)
                    new.append(cnts[e] + c[15])
                key_v[pl.ds((v - v0) * 16, 16)] = keyv
                return tuple(new)

            lax.fori_loop(v0, v0 + nvt, body, cnts0, unroll=False)
            pltpu.sync_copy(key_v, key_hbm.at[pl.ds(v0 * 16, nvt * 16)])

    return keys_k(s_e)


def _sc_dispatch_xg(xt, k0_2d, k1_2d, E, cap):
    T, D = xt.shape
    t_per_w = T // _NW
    TB = 32
    nch = t_per_w // TB
    mesh = plsc.VectorSubcoreMesh(core_axis_name="c", subcore_axis_name="s")

    @functools.partial(
        pl.kernel, mesh=mesh,
        out_type=jax.ShapeDtypeStruct(((E + 1) * cap, D), jnp.float32),
        scratch_types=[
            pltpu.VMEM((nch, TB), jnp.int32),
            pltpu.VMEM((nch, TB), jnp.int32),
            pltpu.VMEM((TB, D), jnp.float32),
            pltpu.SemaphoreType.DMA,
            pltpu.SemaphoreType.DMA,
        ],
    )
    def disp_k(xt_hbm, k0_hbm, k1_hbm, xg_hbm, idx_a, idx_b, buf, sem_a, sem_b):
        wid = lax.axis_index("s") * _NC + lax.axis_index("c")
        base = wid * t_per_w
        pltpu.sync_copy(k0_hbm.at[pl.ds(wid * nch, nch)], idx_a)
        pltpu.sync_copy(k1_hbm.at[pl.ds(wid * nch, nch)], idx_b)

        def chunk(i, carry):
            pltpu.sync_copy(xt_hbm.at[pl.ds(base + i * TB, TB)], buf)
            cp_a = pltpu.async_copy(buf, xg_hbm.at[idx_a.at[i]], sem_a)
            cp_b = pltpu.async_copy(buf, xg_hbm.at[idx_b.at[i]], sem_b)
            cp_a.wait()
            cp_b.wait()
            return carry

        lax.fori_loop(0, nch, chunk, 0, unroll=False)

    return disp_k(xt, k0_2d, k1_2d)


def _ffn(xg, w1, w2, cap):
    D = xg.shape[1]
    E, _, F = w1.shape
    BF = 512
    nf = F // BF

    def body(xg_ref, w1_ref, w2_ref, o_ref):
        e = pl.program_id(0)
        f = pl.program_id(1)

        @pl.when(f == 0)
        def _():
            o_ref[...] = jnp.zeros_like(o_ref)

        @pl.when(e < E)
        def _():
            h = jnp.dot(xg_ref[...], w1_ref[0], preferred_element_type=jnp.float32)
            h = h * 0.5 * (1.0 + lax.erf(h * 0.7071067811865476))
            o_ref[...] += jnp.dot(h, w2_ref[0], preferred_element_type=jnp.float32)

    clamp = lambda e: jnp.minimum(e, E - 1)
    return pl.pallas_call(
        body,
        grid=(E + 1, nf),
        in_specs=[
            pl.BlockSpec((cap, D), lambda e, f: (clamp(e), 0)),
            pl.BlockSpec((1, D, BF), lambda e, f: (clamp(e), 0, jnp.where(e < E, f, 0))),
            pl.BlockSpec((1, BF, D), lambda e, f: (clamp(e), jnp.where(e < E, f, 0), 0)),
        ],
        out_specs=pl.BlockSpec((cap, D), lambda e, f: (e, 0)),
        out_shape=jax.ShapeDtypeStruct(((E + 1) * cap, D), jnp.float32),
    )(xg, w1, w2)


def _sc_combine(o, k0, k1, w0, w1):
    T = k0.shape[0]
    D = o.shape[1]
    t_per_w = T // _NW
    TB = 32
    mesh = plsc.VectorSubcoreMesh(core_axis_name="c", subcore_axis_name="s")

    @functools.partial(
        pl.kernel, mesh=mesh,
        out_type=jax.ShapeDtypeStruct((T, D), jnp.float32),
        scratch_types=[
            pltpu.VMEM((t_per_w,), jnp.int32),
            pltpu.VMEM((t_per_w,), jnp.int32),
            pltpu.VMEM((t_per_w,), jnp.float32),
            pltpu.VMEM((t_per_w,), jnp.float32),
            pltpu.VMEM((TB, D), jnp.float32),
            pltpu.VMEM((TB, D), jnp.float32),
            pltpu.VMEM((TB, D), jnp.float32),
            pltpu.SemaphoreType.DMA,
            pltpu.SemaphoreType.DMA,
        ],
    )
    def combine_k(o_hbm, k0_hbm, k1_hbm, w0_hbm, w1_hbm, out_hbm,
                  k0_v, k1_v, w0_v, w1_v, buf_a, buf_b, y_v, sem_a, sem_b):
        wid = lax.axis_index("s") * _NC + lax.axis_index("c")
        base = wid * t_per_w
        pltpu.sync_copy(k0_hbm.at[pl.ds(base, t_per_w)], k0_v)
        pltpu.sync_copy(k1_hbm.at[pl.ds(base, t_per_w)], k1_v)
        pltpu.sync_copy(w0_hbm.at[pl.ds(base, t_per_w)], w0_v)
        pltpu.sync_copy(w1_hbm.at[pl.ds(base, t_per_w)], w1_v)
        dn = lax.GatherDimensionNumbers(offset_dims=(), collapsed_slice_dims=(0,),
                                        start_index_map=(0,))

        def bcast(wv, lane):
            idx = jnp.zeros((16,), jnp.int32) + lane
            return lax.gather(wv, idx[:, None], dn, slice_sizes=(1,),
                              mode=lax.GatherScatterMode.PROMISE_IN_BOUNDS)

        def chunk(i, carry):
            cp_a = pltpu.async_copy(o_hbm.at[k0_v.at[pl.ds(i * TB, TB)]], buf_a, sem_a)
            cp_b = pltpu.async_copy(o_hbm.at[k1_v.at[pl.ds(i * TB, TB)]], buf_b, sem_b)
            cp_a.wait()
            cp_b.wait()

            def per_token(j, c2):
                g16 = (j // 16) * 16
                wva = w0_v[pl.ds(i * TB + g16, 16)]
                wvb = w1_v[pl.ds(i * TB + g16, 16)]
                lane = j - g16
                wa = bcast(wva, lane)
                wb = bcast(wvb, lane)

                def per_lane(g, c3):
                    sl = pl.ds(g * 16, 16)
                    y_v[j, sl] = buf_a[j, sl] * wa + buf_b[j, sl] * wb
                    return c3

                return lax.fori_loop(0, D // 16, per_lane, c2, unroll=4)

            lax.fori_loop(0, TB, per_token, 0, unroll=False)
            pltpu.sync_copy(y_v, out_hbm.at[pl.ds(base + i * TB, TB)])
            return carry

        lax.fori_loop(0, t_per_w // TB, chunk, 0, unroll=False)

    return combine_k(o, k0, k1, w0, w1)


def kernel(x, wr, w1, w2):
    Bq, Sq, D = x.shape
    E = w1.shape[0]
    T = Bq * Sq
    cap = int(_CAP_FACTOR * _TOP_K * T / E)
    xt = jnp.transpose(x, (1, 0, 2)).reshape(T, D)
    s_e, s_w = _tc_router(xt, wr)
    key = _sc_keys(s_e, E, cap)
    k0 = key[0::2]
    k1 = key[1::2]
    w0 = s_w[0::2]
    w1s = s_w[1::2]
    TB = 32
    xg = _sc_dispatch_xg(xt, k0.reshape(T // TB, TB), k1.reshape(T // TB, TB),
                         E, cap)
    o = _ffn(xg, w1, w2, cap)
    y = _sc_combine(o, k0, k1, w0, w1s)
    return jnp.transpose(y.reshape(Sq, Bq, D), (1, 0, 2))


# speedup vs baseline: 1.0108x; 1.0008x over previous
import functools

import jax
import jax.numpy as jnp
from jax import lax
from jax.experimental import pallas as pl
from jax.experimental.pallas import tpu as pltpu
from jax.experimental.pallas import tpu_sc as plsc

_TOP_K = 2
_CAP_FACTOR = 1.0

_info = plsc.get_sparse_core_info()
_NC = _info.num_cores
_NS = _info.num_subcores
_NW = _NC * _NS


def _tc_router(xt, wr):
    T, D = xt.shape
    E = wr.shape[1]
    BT = 512
    wrp = jnp.pad(wr, ((0, 0), (0, 128 - E)))

    def body(x_ref, wr_ref, oi_ref, ow_ref):
        l = jnp.dot(x_ref[...], wr_ref[...], preferred_element_type=jnp.float32)
        lane = lax.broadcasted_iota(jnp.int32, l.shape, 1)
        ninf = jnp.float32(-1e30)
        lm = jnp.where(lane < E, l, ninf)
        m1 = jnp.max(lm, axis=1, keepdims=True)
        i1 = jnp.min(jnp.where(lm == m1, lane, 128), axis=1, keepdims=True)
        lm2 = jnp.where(lane == i1, ninf, lm)
        m2 = jnp.max(lm2, axis=1, keepdims=True)
        i2 = jnp.min(jnp.where(lm2 == m2, lane, 128), axis=1, keepdims=True)
        z = jnp.sum(jnp.exp(lm - m1), axis=1, keepdims=True)
        w1v = 1.0 / z
        w2v = jnp.exp(m2 - m1) / z
        oi_ref[...] = jnp.where(lane == 0, i1, jnp.where(lane == 1, i2, 0))
        ow_ref[...] = jnp.where(lane == 0, w1v, jnp.where(lane == 1, w2v, 0.0))

    oi, ow = pl.pallas_call(
        body,
        grid=(T // BT,),
        in_specs=[
            pl.BlockSpec((BT, D), lambda i: (i, 0)),
            pl.BlockSpec((D, 128), lambda i: (0, 0)),
        ],
        out_specs=[
            pl.BlockSpec((BT, 128), lambda i: (i, 0)),
            pl.BlockSpec((BT, 128), lambda i: (i, 0)),
        ],
        out_shape=[
            jax.ShapeDtypeStruct((T, 128), jnp.int32),
            jax.ShapeDtypeStruct((T, 128), jnp.float32),
        ],
    )(xt, wrp)
    return oi[:, :_TOP_K].reshape(-1), ow[:, :_TOP_K].reshape(-1)


def _prefix16(m):
    iota = lax.iota(jnp.int32, 16)
    c = jnp.where(m, 1, 0)
    for k in (1, 2, 4, 8):
        idx = jnp.maximum(iota - k, 0)
        g = lax.gather(
            c, idx[:, None],
            lax.GatherDimensionNumbers(offset_dims=(), collapsed_slice_dims=(0,),
                                       start_index_map=(0,)),
            slice_sizes=(1,), mode=lax.GatherScatterMode.PROMISE_IN_BOUNDS)
        c = c + jnp.where(iota >= k, g, 0)
    return c


def _sc_keys(s_e, E, cap):
    S = s_e.shape[0]
    NV = S // 16
    R = E * cap
    NT = 16
    nvt = NV // NT
    mesh = plsc.VectorSubcoreMesh(core_axis_name="c", subcore_axis_name="s")

    @functools.partial(
        pl.kernel, mesh=mesh,
        out_type=jax.ShapeDtypeStruct((S,), jnp.int32),
        scratch_types=[
            pltpu.VMEM((S,), jnp.int32),
            pltpu.VMEM((S // NT,), jnp.int32),
        ],
    )
    def keys_k(se_hbm, key_hbm, se_v, key_v):
        role = lax.axis_index("s") * _NC + lax.axis_index("c")

        @pl.when(role < NT)
        def _():
            v0 = role * nvt
            pltpu.sync_copy(se_hbm, se_v)

            def pcnt(v, cnts):
                se = se_v[pl.ds(v * 16, 16)]
                return tuple(
                    cnts[e] + _prefix16(se == e)[15] for e in range(E))

            cnts0 = lax.fori_loop(0, v0, pcnt, (jnp.int32(0),) * E,
                                  unroll=False)

            def body(v, cnts):
                se = se_v[pl.ds(v * 16, 16)]
                keyv = jnp.full((16,), R, jnp.int32)
                new = []
                for e in range(E):
                    m = se == e
                    c = _prefix16(m)
                    pos = cnts[e] + c - 1
                    val = m & (pos < cap)
                    keyv = jnp.where(val, e * cap + pos, keyv)
                    new.append(cnts[e] + c[15])
                key_v[pl.ds((v - v0) * 16, 16)] = keyv
                return tuple(new)

            lax.fori_loop(v0, v0 + nvt, body, cnts0, unroll=False)
            pltpu.sync_copy(key_v, key_hbm.at[pl.ds(v0 * 16, nvt * 16)])

    return keys_k(s_e)


def _sc_dispatch_xg(xt, k0_2d, k1_2d, E, cap):
    T, D = xt.shape
    t_per_w = T // _NW
    TB = 32
    nch = t_per_w // TB
    mesh = plsc.VectorSubcoreMesh(core_axis_name="c", subcore_axis_name="s")

    @functools.partial(
        pl.kernel, mesh=mesh,
        out_type=jax.ShapeDtypeStruct(((E + 1) * cap, D), jnp.float32),
        scratch_types=[
            pltpu.VMEM((nch, TB), jnp.int32),
            pltpu.VMEM((nch, TB), jnp.int32),
            pltpu.VMEM((TB, D), jnp.float32),
            pltpu.SemaphoreType.DMA,
            pltpu.SemaphoreType.DMA,
        ],
    )
    def disp_k(xt_hbm, k0_hbm, k1_hbm, xg_hbm, idx_a, idx_b, buf, sem_a, sem_b):
        wid = lax.axis_index("s") * _NC + lax.axis_index("c")
        base = wid * t_per_w
        pltpu.sync_copy(k0_hbm.at[pl.ds(wid * nch, nch)], idx_a)
        pltpu.sync_copy(k1_hbm.at[pl.ds(wid * nch, nch)], idx_b)

        def chunk(i, carry):
            pltpu.sync_copy(xt_hbm.at[pl.ds(base + i * TB, TB)], buf)
            cp_a = pltpu.async_copy(buf, xg_hbm.at[idx_a.at[i]], sem_a)
            cp_b = pltpu.async_copy(buf, xg_hbm.at[idx_b.at[i]], sem_b)
            cp_a.wait()
            cp_b.wait()
            return carry

        lax.fori_loop(0, nch, chunk, 0, unroll=False)

    return disp_k(xt, k0_2d, k1_2d)


def _ffn(xg, w1, w2, cap):
    D = xg.shape[1]
    E, _, F = w1.shape
    BF = 512
    nf = F // BF

    def body(xg_ref, w1_ref, w2_ref, o_ref):
        e = pl.program_id(0)
        f = pl.program_id(1)

        @pl.when(f == 0)
        def _():
            o_ref[...] = jnp.zeros_like(o_ref)

        @pl.when(e < E)
        def _():
            h = jnp.dot(xg_ref[...], w1_ref[0], preferred_element_type=jnp.float32)
            h = h * 0.5 * (1.0 + lax.erf(h * 0.7071067811865476))
            o_ref[...] += jnp.dot(h, w2_ref[0], preferred_element_type=jnp.float32)

    clamp = lambda e: jnp.minimum(e, E - 1)
    return pl.pallas_call(
        body,
        grid=(E + 1, nf),
        in_specs=[
            pl.BlockSpec((cap, D), lambda e, f: (clamp(e), 0)),
            pl.BlockSpec((1, D, BF), lambda e, f: (clamp(e), 0, jnp.where(e < E, f, 0))),
            pl.BlockSpec((1, BF, D), lambda e, f: (clamp(e), jnp.where(e < E, f, 0), 0)),
        ],
        out_specs=pl.BlockSpec((cap, D), lambda e, f: (e, 0)),
        out_shape=jax.ShapeDtypeStruct(((E + 1) * cap, D), jnp.float32),
    )(xg, w1, w2)


def _sc_combine(o, k0, k1, w0, w1):
    T = k0.shape[0]
    D = o.shape[1]
    t_per_w = T // _NW
    TB = 32
    mesh = plsc.VectorSubcoreMesh(core_axis_name="c", subcore_axis_name="s")

    @functools.partial(
        pl.kernel, mesh=mesh,
        out_type=jax.ShapeDtypeStruct((T, D), jnp.float32),
        scratch_types=[
            pltpu.VMEM((t_per_w,), jnp.int32),
            pltpu.VMEM((t_per_w,), jnp.int32),
            pltpu.VMEM((t_per_w,), jnp.float32),
            pltpu.VMEM((t_per_w,), jnp.float32),
            pltpu.VMEM((TB, D), jnp.float32),
            pltpu.VMEM((TB, D), jnp.float32),
            pltpu.VMEM((TB, D), jnp.float32),
            pltpu.SemaphoreType.DMA,
            pltpu.SemaphoreType.DMA,
        ],
    )
    def combine_k(o_hbm, k0_hbm, k1_hbm, w0_hbm, w1_hbm, out_hbm,
                  k0_v, k1_v, w0_v, w1_v, buf_a, buf_b, y_v, sem_a, sem_b):
        wid = lax.axis_index("s") * _NC + lax.axis_index("c")
        base = wid * t_per_w
        pltpu.sync_copy(k0_hbm.at[pl.ds(base, t_per_w)], k0_v)
        pltpu.sync_copy(k1_hbm.at[pl.ds(base, t_per_w)], k1_v)
        pltpu.sync_copy(w0_hbm.at[pl.ds(base, t_per_w)], w0_v)
        pltpu.sync_copy(w1_hbm.at[pl.ds(base, t_per_w)], w1_v)
        dn = lax.GatherDimensionNumbers(offset_dims=(), collapsed_slice_dims=(0,),
                                        start_index_map=(0,))

        def bcast(wv, lane):
            idx = jnp.zeros((16,), jnp.int32) + lane
            return lax.gather(wv, idx[:, None], dn, slice_sizes=(1,),
                              mode=lax.GatherScatterMode.PROMISE_IN_BOUNDS)

        def chunk(i, carry):
            cp_a = pltpu.async_copy(o_hbm.at[k0_v.at[pl.ds(i * TB, TB)]], buf_a, sem_a)
            cp_b = pltpu.async_copy(o_hbm.at[k1_v.at[pl.ds(i * TB, TB)]], buf_b, sem_b)
            cp_a.wait()
            cp_b.wait()

            def per_token(j, c2):
                g16 = (j // 16) * 16
                wva = w0_v[pl.ds(i * TB + g16, 16)]
                wvb = w1_v[pl.ds(i * TB + g16, 16)]
                lane = j - g16
                wa = bcast(wva, lane)
                wb = bcast(wvb, lane)

                def per_lane(g, c3):
                    sl = pl.ds(g * 16, 16)
                    y_v[j, sl] = buf_a[j, sl] * wa + buf_b[j, sl] * wb
                    return c3

                return lax.fori_loop(0, D // 16, per_lane, c2, unroll=16)

            lax.fori_loop(0, TB, per_token, 0, unroll=2)
            pltpu.sync_copy(y_v, out_hbm.at[pl.ds(base + i * TB, TB)])
            return carry

        lax.fori_loop(0, t_per_w // TB, chunk, 0, unroll=False)

    return combine_k(o, k0, k1, w0, w1)


def kernel(x, wr, w1, w2):
    Bq, Sq, D = x.shape
    E = w1.shape[0]
    T = Bq * Sq
    cap = int(_CAP_FACTOR * _TOP_K * T / E)
    xt = jnp.transpose(x, (1, 0, 2)).reshape(T, D)
    s_e, s_w = _tc_router(xt, wr)
    key = _sc_keys(s_e, E, cap)
    k0 = key[0::2]
    k1 = key[1::2]
    w0 = s_w[0::2]
    w1s = s_w[1::2]
    TB = 32
    xg = _sc_dispatch_xg(xt, k0.reshape(T // TB, TB), k1.reshape(T // TB, TB),
                         E, cap)
    o = _ffn(xg, w1, w2, cap)
    y = _sc_combine(o, k0, k1, w0, w1s)
    return jnp.transpose(y.reshape(Sq, Bq, D), (1, 0, 2))
